# baseline (speedup 1.0000x reference)
import jax
import jax.numpy as jnp
import numpy as np
from jax.experimental import pallas as pl
from jax.experimental.pallas import tpu as pltpu

_B, _N, _C = 32, 20000, 21
_NB = 5000
_NCHUNK = _N // _NB
_NBLK = _B * _NCHUNK
_RATIO = 4.0
_HARD = 100.0

_DN = (((1,), (1,)), ((), ()))
_DR = (((1,), (0,)), ((), ()))


def _wmats():
    c = np.arange(_C)
    w_psel = np.zeros((8, _C), np.float32)
    w_conf = np.zeros((8, _C), np.float32)
    w_t0 = np.zeros((8, _C), np.float32)
    w_psel[0, :] = 1.0
    w_conf[1, :] = (c >= 1)
    w_t0[2, 0] = 1.0
    return jnp.asarray(w_psel), jnp.asarray(w_conf), jnp.asarray(w_t0)


def _body(yp0_ref, yp1_ref, yt0_ref, yt1_ref, wp_ref, wc_ref, wt_ref,
          o64_ref, on_ref, out_ref, vi_s, w_s, acc_ref):
    b = pl.program_id(0)
    j = pl.program_id(1)

    @pl.when((b == 0) & (j == 0))
    def _init():
        def z(t, carry):
            acc_ref[t] = 0.0
            return carry
        jax.lax.fori_loop(0, _B, z, 0)

    wp = wp_ref[...].astype(jnp.bfloat16)
    wc = wc_ref[...].astype(jnp.bfloat16)
    wt = wt_ref[...].astype(jnp.bfloat16)

    for h, (yp_ref, yt_ref) in enumerate(((yp0_ref, yt0_ref),
                                          (yp1_ref, yt1_ref))):
        yp = yp_ref[0]
        yt = yt_ref[0]
        ypc = jnp.maximum(yp, 1e-7)
        m = (yt * ypc).astype(jnp.bfloat16)
        ypc_hi = ypc.astype(jnp.bfloat16)
        ypc_lo = (ypc - ypc_hi.astype(jnp.float32)).astype(jnp.bfloat16)
        yt16 = yt.astype(jnp.bfloat16)
        z = (jax.lax.dot_general(wp, m, _DN,
                                 preferred_element_type=jnp.float32)
             + jax.lax.dot_general(wc, ypc_hi, _DN,
                                   preferred_element_type=jnp.float32)
             + jax.lax.dot_general(wc, ypc_lo, _DN,
                                   preferred_element_type=jnp.float32)
             + jax.lax.dot_general(wt, yt16, _DN,
                                   preferred_element_type=jnp.float32))
        psel = z[0:1, :]
        conf = z[1:2, :]
        t0 = z[2:3, :]
        cls = -jnp.log(psel)
        v = conf * t0
        r = b * _NCHUNK + j * 2 + h
        vi_s[r] = jax.lax.bitcast_convert_type(v, jnp.int32)
        w_s[r] = cls
        acc_ref[b] = acc_ref[b] + (float(_NB) - jnp.sum(t0))

    @pl.when((b == _B - 1) & (j == _NCHUNK // 2 - 1))
    def _final():
        vi = vi_s[...]
        w = w_s[...]
        o64 = o64_ref[...]
        on = on_ref[...]

        def gsum_cnt(x):
            p = jax.lax.dot_general(o64, x, _DR,
                                    preferred_element_type=jnp.float32)
            q = jax.lax.dot_general(p.astype(jnp.bfloat16), on, _DR,
                                    preferred_element_type=jnp.float32)
            return q[0, 0]

        def gsum_w(x):
            return jnp.sum(jnp.sum(x, axis=0))

        vi2 = vi[:, 0, :]
        w2 = w[:, 0, :]
        one = jnp.float32(1.0)
        zero = jnp.float32(0.0)
        pos_sum = gsum_w(jnp.where(vi2 == 0, w2, zero))

        def batch_stats(bb, carry):
            kf, denom = carry
            npb = acc_ref[bb]
            nn = jnp.minimum(_RATIO * npb, float(_N) - npb)
            return kf + nn, denom + jnp.maximum(npb, 1.0)

        kf, denom = jax.lax.fori_loop(
            0, _B, batch_stats, (jnp.float32(0.0), jnp.float32(0.0)))
        kf = jnp.where(kf > 0.0, kf, _HARD)
        k = kf.astype(jnp.int32).astype(jnp.float32)

        def bis(_, lohi):
            lo, hi = lohi
            mid = (lo + hi) // 2
            c = gsum_cnt(jnp.where(vi2 > mid, one, zero))
            big = c >= k
            return jnp.where(big, mid, lo), jnp.where(big, hi, mid)

        lo0 = jnp.int32(0x35000000)
        hi0 = jnp.int32(0x3F910000)
        _, hi = jax.lax.fori_loop(0, 28, bis, (lo0, hi0))
        gt = vi2 > hi
        eq = vi2 == hi
        cnt_gt = gsum_cnt(jnp.where(gt, one, zero))
        neg_gt = gsum_w(jnp.where(gt, w2, zero))
        tie_sum = gsum_w(jnp.where(eq, w2, zero))
        tie_cnt = gsum_cnt(jnp.where(eq, one, zero))
        neg = neg_gt + (k - cnt_gt) * tie_sum / jnp.maximum(tie_cnt, 1.0)
        out_ref[0] = (pos_sum + neg) / denom


def kernel(y_pred, y_true):
    w_psel, w_conf, w_t0 = _wmats()
    o64 = jnp.ones((8, _NBLK), jnp.float32)
    on = jnp.ones((_NB, 128), jnp.bfloat16)
    wspec = pl.BlockSpec((8, _C), lambda b, j: (0, 0))
    qrt = lambda qq: pl.BlockSpec((1, _NB, _C), lambda b, j: (b, j * 2 + qq, 0))
    out = pl.pallas_call(
        _body,
        grid=(_B, _NCHUNK // 2),
        in_specs=[
            qrt(0),
            qrt(1),
            qrt(0),
            qrt(1),
            wspec,
            wspec,
            wspec,
            pl.BlockSpec((8, _NBLK), lambda b, j: (0, 0)),
            pl.BlockSpec((_NB, 128), lambda b, j: (0, 0)),
        ],
        out_specs=pl.BlockSpec(memory_space=pltpu.SMEM),
        out_shape=jax.ShapeDtypeStruct((1,), jnp.float32),
        scratch_shapes=[
            pltpu.VMEM((_NBLK, 1, _NB), jnp.int32),
            pltpu.VMEM((_NBLK, 1, _NB), jnp.float32),
            pltpu.SMEM((_B,), jnp.float32),
        ],
    )(y_pred, y_pred, y_true, y_true, w_psel, w_conf, w_t0, o64, on)
    return jnp.reshape(out, ())
